# baseline (speedup 1.0000x reference)
import functools

import jax
import jax.numpy as jnp
from jax import lax
from jax.experimental import pallas as pl
from jax.experimental.pallas import tpu as pltpu
from jax.experimental.pallas import tpu_sc as plsc

N_EDGES = 3_200_000
NUM_TYPES = 64
EMB = 16
LANES = 16

NC = 2
NS = 16
NW = NC * NS
CHUNK = 1024
N_CHUNKS = N_EDGES // CHUNK
BASE_CHUNKS = N_CHUNKS // NW
EXTRA_W = N_CHUNKS - BASE_CHUNKS * NW
GROUPS = CHUNK // LANES
PAIRS = (BASE_CHUNKS - 1) // 2

_mesh = plsc.VectorSubcoreMesh(core_axis_name="c", subcore_axis_name="s")


@functools.partial(
    pl.kernel,
    mesh=_mesh,
    out_type=jax.ShapeDtypeStruct((EMB, N_EDGES), jnp.float32),
    scratch_types=[
        pltpu.VMEM((NUM_TYPES * EMB,), jnp.float32),
        pltpu.VMEM((CHUNK,), jnp.int32),
        pltpu.VMEM((CHUNK,), jnp.int32),
        pltpu.VMEM((EMB, CHUNK), jnp.float32),
        pltpu.VMEM((EMB, CHUNK), jnp.float32),
        pltpu.SemaphoreType.DMA,
        pltpu.SemaphoreType.DMA,
        pltpu.SemaphoreType.DMA,
        pltpu.SemaphoreType.DMA,
    ],
    compiler_params=pltpu.CompilerParams(needs_layout_passes=False),
)
def _emb_kernel(
    idx_hbm, table_hbm, out_hbm,
    table_v, idx_v0, idx_v1, out_v0, out_v1, isem0, isem1, osem0, osem1,
):
    wid = lax.axis_index("s") * NC + lax.axis_index("c")
    start = wid * BASE_CHUNKS + jnp.minimum(wid, EXTRA_W)
    extra = wid < EXTRA_W

    idx_bufs = (idx_v0, idx_v1)
    out_bufs = (out_v0, out_v1)
    isems = (isem0, isem1)
    osems = (osem0, osem1)

    pltpu.sync_copy(table_hbm, table_v)

    def idx_start(i, b):
        base = (start + i) * CHUNK
        pltpu.async_copy(idx_hbm.at[pl.ds(base, CHUNK)], idx_bufs[b], isems[b])

    def idx_wait(b):
        pltpu.make_async_copy(
            idx_hbm.at[pl.ds(0, CHUNK)], idx_bufs[b], isems[b]
        ).wait()

    def out_start(i, b):
        base = (start + i) * CHUNK
        pltpu.async_copy(
            out_bufs[b], out_hbm.at[:, pl.ds(base, CHUNK)], osems[b]
        )

    def out_wait(b):
        pltpu.make_async_copy(
            out_bufs[b], out_hbm.at[:, pl.ds(0, CHUNK)], osems[b]
        ).wait()

    jlane = lax.iota(jnp.int32, LANES)

    def compute(b):
        idx_v = idx_bufs[b]
        out_v = out_bufs[b]

        def group_body(g, carry2):
            o = g * (8 * LANES)
            fs = [idx_v[pl.ds(o + k * LANES, LANES)] for k in range(8)]
            pipe = [
                [plsc.load_gather(table_v, [f]) for f in fs]
            ]
            for c in range(EMB):
                if c + 1 < EMB:
                    pipe.append([
                        plsc.load_gather(table_v, [f + (c + 1) * NUM_TYPES])
                        for f in fs
                    ])
                vals = pipe.pop(0)
                for k in range(8):
                    out_v[c, pl.ds(o + k * LANES, LANES)] = vals[k]
            return carry2

        lax.fori_loop(0, GROUPS // 8, group_body, 0)

    def step(i, b):
        @pl.when((i + 1 < BASE_CHUNKS) | extra)
        def _():
            idx_start(i + 1, b ^ 1)

        idx_wait(b)

        @pl.when(i >= 2)
        def _():
            out_wait(b)

        compute(b)
        out_start(i, b)

    idx_start(0, 0)

    def pair_body(k, carry):
        step(2 * k, 0)
        step(2 * k + 1, 1)
        return carry

    lax.fori_loop(0, PAIRS, pair_body, 0)

    last = BASE_CHUNKS - 1

    @pl.when(extra)
    def _():
        idx_start(last + 1, 1)

    idx_wait(0)
    out_wait(0)
    compute(0)
    out_start(last, 0)

    out_wait(1)

    @pl.when(extra)
    def _():
        idx_wait(1)
        compute(1)
        out_start(last + 1, 1)

    out_wait(0)

    @pl.when(extra)
    def _():
        out_wait(1)


def kernel(edge_attr, table):
    return _emb_kernel(edge_attr, table.T.reshape(-1)).T


# speedup vs baseline: 1.0549x; 1.0549x over previous
import functools

import jax
import jax.numpy as jnp
from jax import lax
from jax.experimental import pallas as pl
from jax.experimental.pallas import tpu as pltpu
from jax.experimental.pallas import tpu_sc as plsc

N_EDGES = 3_200_000
NUM_TYPES = 64
EMB = 16
LANES = 16

NC = 2
NS = 16
NW = NC * NS
CHUNK = 1024
N_CHUNKS = N_EDGES // CHUNK
BASE_CHUNKS = N_CHUNKS // NW
EXTRA_W = N_CHUNKS - BASE_CHUNKS * NW
GROUPS = CHUNK // LANES
PAIRS = (BASE_CHUNKS - 1) // 2

_mesh = plsc.VectorSubcoreMesh(core_axis_name="c", subcore_axis_name="s")


@functools.partial(
    pl.kernel,
    mesh=_mesh,
    out_type=jax.ShapeDtypeStruct((EMB, N_EDGES), jnp.float32),
    scratch_types=[
        pltpu.VMEM((NUM_TYPES * EMB,), jnp.float32),
        pltpu.VMEM((CHUNK,), jnp.int32),
        pltpu.VMEM((CHUNK,), jnp.int32),
        pltpu.VMEM((EMB, CHUNK), jnp.float32),
        pltpu.VMEM((EMB, CHUNK), jnp.float32),
        pltpu.SemaphoreType.DMA,
        pltpu.SemaphoreType.DMA,
        pltpu.SemaphoreType.DMA,
        pltpu.SemaphoreType.DMA,
    ],
    compiler_params=pltpu.CompilerParams(needs_layout_passes=False),
)
def _emb_kernel(
    idx_hbm, table_hbm, out_hbm,
    table_v, idx_v0, idx_v1, out_v0, out_v1, isem0, isem1, osem0, osem1,
):
    wid = lax.axis_index("s") * NC + lax.axis_index("c")
    start = wid * BASE_CHUNKS + jnp.minimum(wid, EXTRA_W)
    extra = wid < EXTRA_W

    idx_bufs = (idx_v0, idx_v1)
    out_bufs = (out_v0, out_v1)
    isems = (isem0, isem1)
    osems = (osem0, osem1)

    pltpu.sync_copy(table_hbm, table_v)

    def idx_start(i, b):
        base = (start + i) * CHUNK
        pltpu.async_copy(idx_hbm.at[pl.ds(base, CHUNK)], idx_bufs[b], isems[b])

    def idx_wait(b):
        pltpu.make_async_copy(
            idx_hbm.at[pl.ds(0, CHUNK)], idx_bufs[b], isems[b]
        ).wait()

    def out_start(i, b):
        base = (start + i) * CHUNK
        pltpu.async_copy(
            out_bufs[b], out_hbm.at[:, pl.ds(base, CHUNK)], osems[b]
        )

    def out_wait(b):
        pltpu.make_async_copy(
            out_bufs[b], out_hbm.at[:, pl.ds(0, CHUNK)], osems[b]
        ).wait()

    jlane = lax.iota(jnp.int32, LANES)

    def compute(b):
        idx_v = idx_bufs[b]
        out_v = out_bufs[b]

        def group_body(g, carry2):
            o = g * (4 * LANES)
            fs = [idx_v[pl.ds(o + k * LANES, LANES)] for k in range(4)]
            pipe = [
                [plsc.load_gather(table_v, [f]) for f in fs]
            ]
            for c in range(EMB):
                if c + 1 < EMB:
                    pipe.append([
                        plsc.load_gather(table_v, [f + (c + 1) * NUM_TYPES])
                        for f in fs
                    ])
                vals = pipe.pop(0)
                for k in range(4):
                    out_v[c, pl.ds(o + k * LANES, LANES)] = vals[k]
            return carry2

        lax.fori_loop(0, GROUPS // 4, group_body, 0, unroll=2)

    def step(i, b):
        @pl.when((i + 1 < BASE_CHUNKS) | extra)
        def _():
            idx_start(i + 1, b ^ 1)

        idx_wait(b)

        @pl.when(i >= 2)
        def _():
            out_wait(b)

        compute(b)
        out_start(i, b)

    idx_start(0, 0)

    def pair_body(k, carry):
        step(2 * k, 0)
        step(2 * k + 1, 1)
        return carry

    lax.fori_loop(0, PAIRS, pair_body, 0)

    last = BASE_CHUNKS - 1

    @pl.when(extra)
    def _():
        idx_start(last + 1, 1)

    idx_wait(0)
    out_wait(0)
    compute(0)
    out_start(last, 0)

    out_wait(1)

    @pl.when(extra)
    def _():
        idx_wait(1)
        compute(1)
        out_start(last + 1, 1)

    out_wait(0)

    @pl.when(extra)
    def _():
        out_wait(1)


def kernel(edge_attr, table):
    return _emb_kernel(edge_attr, table.T.reshape(-1)).T
